# baseline (speedup 1.0000x reference)
import jax
import jax.numpy as jnp
from jax.experimental import pallas as pl
from jax.experimental.pallas import tpu as pltpu

_XROWS, _XCOLS = 262144, 256
_ZROWS, _ZCOLS = 16384, 1024
_NX = 8
_NZ = 4


def _copy_body(x_ref, y_ref, z_ref, w_ref, xo_ref, zo_ref, zbuf, r45, sems, fsem):
    xc = _XROWS // _NX
    zc = _ZROWS // _NZ
    copies = []
    for i in range(_NX):
        c = pltpu.make_async_copy(
            x_ref.at[pl.ds(i * xc, xc)], xo_ref.at[pl.ds(i * xc, xc)], sems.at[i])
        c.start()
        copies.append(c)
    for j in range(_NZ):
        c = pltpu.make_async_copy(
            z_ref.at[pl.ds(j * zc, zc)], zo_ref.at[pl.ds(j * zc, zc)], sems.at[_NX + j])
        c.start()
        copies.append(c)

    zl = pltpu.make_async_copy(z_ref.at[pl.ds(0, 8)], zbuf, fsem)
    zl.start()
    r45[...] = jnp.full((8, _XCOLS), 45.0, dtype=jnp.float32)
    zl.wait()
    w0, w1, w2 = w_ref[0], w_ref[1], w_ref[2]
    rows = jax.lax.broadcasted_iota(jnp.int32, (8, _ZCOLS), 0)
    cols = jax.lax.broadcasted_iota(jnp.int32, (8, _ZCOLS), 1)
    upd = (jnp.where((rows == 1) & (cols == 3), w0, 0.0)
           + jnp.where((rows == 0) & (cols == 2), w1, 0.0)
           + jnp.where((rows == 0) & (cols == 1), w2, 0.0))
    zbuf[...] = zbuf[...] + upd

    for c in copies:
        c.wait()

    fix = [
        pltpu.make_async_copy(y_ref.at[pl.ds(0, 1)], xo_ref.at[pl.ds(10, 1)], fsem),
        pltpu.make_async_copy(y_ref.at[pl.ds(1, 1)], xo_ref.at[pl.ds(2, 1)], fsem),
        pltpu.make_async_copy(r45.at[pl.ds(0, 1)], xo_ref.at[pl.ds(1, 1)], fsem),
        pltpu.make_async_copy(zbuf.at[pl.ds(0, 2)], zo_ref.at[pl.ds(0, 2)], fsem),
    ]
    for c in fix:
        c.start()
    for c in fix:
        c.wait()


def kernel(x, y, z, w):
    xo, zo = pl.pallas_call(
        _copy_body,
        out_shape=(jax.ShapeDtypeStruct(x.shape, x.dtype),
                   jax.ShapeDtypeStruct(z.shape, z.dtype)),
        in_specs=[pl.BlockSpec(memory_space=pl.ANY),
                  pl.BlockSpec(memory_space=pl.ANY),
                  pl.BlockSpec(memory_space=pl.ANY),
                  pl.BlockSpec(memory_space=pltpu.SMEM)],
        out_specs=(pl.BlockSpec(memory_space=pl.ANY),
                   pl.BlockSpec(memory_space=pl.ANY)),
        scratch_shapes=[
            pltpu.VMEM((8, _ZCOLS), jnp.float32),
            pltpu.VMEM((8, _XCOLS), jnp.float32),
            pltpu.SemaphoreType.DMA((_NX + _NZ,)),
            pltpu.SemaphoreType.DMA,
        ],
    )(x, y, z, w)
    return (xo, zo)
